# Initial kernel scaffold; baseline (speedup 1.0000x reference)
import jax
import jax.numpy as jnp
from jax.experimental import pallas as pl


def kernel(node_vectors, node_features, We1, be1, We2, be2, Wx1, bx1, Wx2, bx2, Wxo, bxo, Winf, binf, Wh1, bh1, Wh2, bh2, Who, bho):
    raise NotImplementedError("write your pallas kernel here")


# speedup vs baseline: 70.8744x; 70.8744x over previous
import functools

import jax
import jax.numpy as jnp
import numpy as np
from jax.experimental import pallas as pl

N = 512
V = 4
C = 3
F = 64
H = 64
BR = 8
E = BR * N
INV_NEIGH = 1.0 / (N - 1)
INV_SQRT_NEIGH = 1.0 / float(np.sqrt(N - 1))


def _silu(x):
    return x * jax.nn.sigmoid(x)


def _prologue_body(f_ref, xflat_ref, w1s_ref, w1r_ref, be1_ref, g_ref,
                   fs_ref, fr_ref, n2_ref):
    f = f_ref[...]
    x = xflat_ref[...]
    fs_ref[...] = jnp.dot(f, w1s_ref[...], preferred_element_type=jnp.float32) + be1_ref[...]
    fr_ref[...] = jnp.dot(f, w1r_ref[...], preferred_element_type=jnp.float32)
    n2_ref[...] = jnp.dot(x * x, g_ref[...], preferred_element_type=jnp.float32)


def _main_body(xflat_ref, xr_ref, n2s_ref, n2r_ref, fs_ref, fr_ref,
               w1v_ref, we2_ref, be2_ref, wx1_ref, bx1_ref, wx2_ref, bx2_ref,
               wxo_ref, bxo_ref, winf_ref, binf_ref, g_ref, gt_ref,
               vec_out_ref, mi_out_ref):
    r0 = pl.program_id(0) * BR
    X = xflat_ref[...]
    xr = xr_ref[...]

    XT = jnp.broadcast_to(X[None], (BR, N, V * C)).reshape(E, V * C)
    xrT = jnp.broadcast_to(xr[:, None, :], (BR, N, V * C)).reshape(E, V * C)

    cross = jnp.dot(XT * xrT, g_ref[...], preferred_element_type=jnp.float32)
    n2sT = jnp.broadcast_to(n2s_ref[...][None], (BR, N, V)).reshape(E, V)
    n2rT = jnp.broadcast_to(n2r_ref[...][:, None, :], (BR, N, V)).reshape(E, V)
    len2 = jnp.maximum(n2sT + n2rT - 2.0 * cross, 0.0)

    FsT = jnp.broadcast_to(fs_ref[...][None], (BR, N, H)).reshape(E, H)
    FrT = jnp.broadcast_to(fr_ref[...][:, None, :], (BR, N, H)).reshape(E, H)
    h = _silu(jnp.dot(len2, w1v_ref[...], preferred_element_type=jnp.float32) + FsT + FrT)
    m = _silu(jnp.dot(h, we2_ref[...], preferred_element_type=jnp.float32) + be2_ref[...])

    px = _silu(jnp.dot(m, wx1_ref[...], preferred_element_type=jnp.float32) + bx1_ref[...])
    px = _silu(jnp.dot(px, wx2_ref[...], preferred_element_type=jnp.float32) + bx2_ref[...])
    po = jnp.dot(px, wxo_ref[...], preferred_element_type=jnp.float32) + bxo_ref[...]
    w = po / (1.0 + jnp.sqrt(len2))

    w12 = jnp.dot(w, gt_ref[...], preferred_element_type=jnp.float32)
    contrib = (w12 * XT).reshape(BR, N, V * C).sum(axis=1)
    wsum = w.reshape(BR, N, V).sum(axis=1)
    shift = jnp.dot(wsum, gt_ref[...], preferred_element_type=jnp.float32) * xr - contrib
    vec_out_ref[...] = xr + shift * INV_NEIGH

    e = jax.nn.sigmoid(jnp.dot(m, winf_ref[...], preferred_element_type=jnp.float32)
                       + binf_ref[...])
    row = jax.lax.broadcasted_iota(jnp.int32, (E, 1), 0)
    jcol = row & (N - 1)
    brow = row >> 9
    mask = (jcol != (r0 + brow)).astype(jnp.float32)
    mi = (m * (e * mask)).reshape(BR, N, H).sum(axis=1)
    mi_out_ref[...] = mi * INV_SQRT_NEIGH


def _epilogue_body(mi_ref, f_ref, wh1a_ref, wh1b_ref, bh1_ref, wh2_ref,
                   bh2_ref, who_ref, bho_ref, out_ref):
    f = f_ref[...]
    ph = _silu(jnp.dot(mi_ref[...], wh1a_ref[...], preferred_element_type=jnp.float32)
               + jnp.dot(f, wh1b_ref[...], preferred_element_type=jnp.float32)
               + bh1_ref[...])
    ph = _silu(jnp.dot(ph, wh2_ref[...], preferred_element_type=jnp.float32) + bh2_ref[...])
    out_ref[...] = jnp.dot(ph, who_ref[...], preferred_element_type=jnp.float32) \
        + bho_ref[...] + f


def _group_sum_matrix():
    g = np.zeros((V * C, V), dtype=np.float32)
    for v in range(V):
        g[v * C:(v + 1) * C, v] = 1.0
    return jnp.asarray(g)


@jax.jit
def kernel(node_vectors, node_features, We1, be1, We2, be2, Wx1, bx1, Wx2, bx2,
           Wxo, bxo, Winf, binf, Wh1, bh1, Wh2, bh2, Who, bho):
    xflat = node_vectors.reshape(N, V * C)
    G = _group_sum_matrix()
    GT = G.T

    fs, fr, n2 = pl.pallas_call(
        _prologue_body,
        out_shape=(
            jax.ShapeDtypeStruct((N, H), jnp.float32),
            jax.ShapeDtypeStruct((N, H), jnp.float32),
            jax.ShapeDtypeStruct((N, V), jnp.float32),
        ),
    )(node_features, xflat, We1[V:V + F], We1[V + F:], be1.reshape(1, H), G)

    full = lambda shape: pl.BlockSpec(shape, lambda i: (0, 0))
    blk = lambda shape: pl.BlockSpec(shape, lambda i: (i, 0))

    vec_out, mi = pl.pallas_call(
        _main_body,
        grid=(N // BR,),
        in_specs=[
            full((N, V * C)),
            blk((BR, V * C)),
            full((N, V)),
            blk((BR, V)),
            full((N, H)),
            blk((BR, H)),
            full((V, H)),
            full((H, H)),
            full((1, H)),
            full((H, H)),
            full((1, H)),
            full((H, H)),
            full((1, H)),
            full((H, V)),
            full((1, V)),
            full((H, 1)),
            full((1, 1)),
            full((V * C, V)),
            full((V, V * C)),
        ],
        out_specs=(
            blk((BR, V * C)),
            blk((BR, H)),
        ),
        out_shape=(
            jax.ShapeDtypeStruct((N, V * C), jnp.float32),
            jax.ShapeDtypeStruct((N, H), jnp.float32),
        ),
    )(xflat, xflat, n2, n2, fs, fr,
      We1[:V], We2, be2.reshape(1, H), Wx1, bx1.reshape(1, H), Wx2,
      bx2.reshape(1, H), Wxo, bxo.reshape(1, V), Winf, binf.reshape(1, 1),
      G, GT)

    features_out = pl.pallas_call(
        _epilogue_body,
        out_shape=jax.ShapeDtypeStruct((N, F), jnp.float32),
    )(mi, node_features, Wh1[:H], Wh1[H:], bh1.reshape(1, H), Wh2,
      bh2.reshape(1, H), Who, bho.reshape(1, F))

    return vec_out.reshape(N, V, C), features_out


# speedup vs baseline: 74.1480x; 1.0462x over previous
import functools

import jax
import jax.numpy as jnp
import numpy as np
from jax.experimental import pallas as pl

N = 512
V = 4
C = 3
F = 64
H = 64
BR = 8
E = BR * N
INV_NEIGH = 1.0 / (N - 1)
INV_SQRT_NEIGH = 1.0 / float(np.sqrt(N - 1))


def _silu(x):
    return x / (1.0 + jnp.exp(-x))


def _prologue_body(f_ref, xflat_ref, w1s_ref, w1r_ref, be1_ref, g_ref,
                   fs_ref, fr_ref, n2_ref):
    f = f_ref[...]
    x = xflat_ref[...]
    fs_ref[...] = jnp.dot(f, w1s_ref[...], preferred_element_type=jnp.float32) + be1_ref[...]
    fr_ref[...] = jnp.dot(f, w1r_ref[...], preferred_element_type=jnp.float32)
    n2_ref[...] = jnp.dot(x * x, g_ref[...], preferred_element_type=jnp.float32)


def _main_body(xflat_ref, xr_ref, n2s_ref, n2r_ref, fs_ref, fr_ref,
               w1v_ref, we2_ref, be2_ref, wx1_ref, bx1_ref, wx2_ref, bx2_ref,
               wxo_ref, bxo_ref, g_ref, gt_ref,
               vec_out_ref, mi_out_ref):
    r0 = pl.program_id(0) * BR
    X = xflat_ref[...]
    xr = xr_ref[...]

    XT = jnp.broadcast_to(X[None], (BR, N, V * C)).reshape(E, V * C)
    xrT = jnp.broadcast_to(xr[:, None, :], (BR, N, V * C)).reshape(E, V * C)

    cross = jnp.dot(XT * xrT, g_ref[...], preferred_element_type=jnp.float32)
    n2sT = jnp.broadcast_to(n2s_ref[...][None], (BR, N, V)).reshape(E, V)
    n2rT = jnp.broadcast_to(n2r_ref[...][:, None, :], (BR, N, V)).reshape(E, V)
    len2 = jnp.maximum(n2sT + n2rT - 2.0 * cross, 0.0)

    FsT = jnp.broadcast_to(fs_ref[...][None], (BR, N, H)).reshape(E, H)
    FrT = jnp.broadcast_to(fr_ref[...][:, None, :], (BR, N, H)).reshape(E, H)
    h = _silu(jnp.dot(len2, w1v_ref[...], preferred_element_type=jnp.float32) + FsT + FrT)
    m = _silu(jnp.dot(h, we2_ref[...], preferred_element_type=jnp.float32) + be2_ref[...])

    pre = jnp.dot(m, wx1_ref[...], preferred_element_type=jnp.float32) + bx1_ref[...]
    sig = 1.0 / (1.0 + jnp.exp(-pre))
    px = pre[:, :H] * sig[:, :H]
    e = sig[:, H:H + 1]
    px = _silu(jnp.dot(px, wx2_ref[...], preferred_element_type=jnp.float32) + bx2_ref[...])
    po = jnp.dot(px, wxo_ref[...], preferred_element_type=jnp.float32) + bxo_ref[...]
    w = po / (1.0 + jnp.sqrt(len2))

    w12 = jnp.dot(w, gt_ref[...], preferred_element_type=jnp.float32)
    contrib = (w12 * XT).reshape(BR, N, V * C).sum(axis=1)
    wsum = w.reshape(BR, N, V).sum(axis=1)
    shift = jnp.dot(wsum, gt_ref[...], preferred_element_type=jnp.float32) * xr - contrib
    vec_out_ref[...] = xr + shift * INV_NEIGH

    row = jax.lax.broadcasted_iota(jnp.int32, (E, 1), 0)
    jcol = row & (N - 1)
    brow = row >> 9
    mask = (jcol != (r0 + brow)).astype(jnp.float32)
    mi = (m * (e * mask)).reshape(BR, N, H).sum(axis=1)
    mi_out_ref[...] = mi * INV_SQRT_NEIGH


def _epilogue_body(mi_ref, f_ref, wh1a_ref, wh1b_ref, bh1_ref, wh2_ref,
                   bh2_ref, who_ref, bho_ref, out_ref):
    f = f_ref[...]
    ph = _silu(jnp.dot(mi_ref[...], wh1a_ref[...], preferred_element_type=jnp.float32)
               + jnp.dot(f, wh1b_ref[...], preferred_element_type=jnp.float32)
               + bh1_ref[...])
    ph = _silu(jnp.dot(ph, wh2_ref[...], preferred_element_type=jnp.float32) + bh2_ref[...])
    out_ref[...] = jnp.dot(ph, who_ref[...], preferred_element_type=jnp.float32) \
        + bho_ref[...] + f


def _group_sum_matrix():
    g = np.zeros((V * C, V), dtype=np.float32)
    for v in range(V):
        g[v * C:(v + 1) * C, v] = 1.0
    return jnp.asarray(g)


@jax.jit
def kernel(node_vectors, node_features, We1, be1, We2, be2, Wx1, bx1, Wx2, bx2,
           Wxo, bxo, Winf, binf, Wh1, bh1, Wh2, bh2, Who, bho):
    xflat = node_vectors.reshape(N, V * C)
    G = _group_sum_matrix()
    GT = G.T

    fs, fr, n2 = pl.pallas_call(
        _prologue_body,
        out_shape=(
            jax.ShapeDtypeStruct((N, H), jnp.float32),
            jax.ShapeDtypeStruct((N, H), jnp.float32),
            jax.ShapeDtypeStruct((N, V), jnp.float32),
        ),
    )(node_features, xflat, We1[V:V + F], We1[V + F:], be1.reshape(1, H), G)

    full = lambda shape: pl.BlockSpec(shape, lambda i: (0, 0))
    blk = lambda shape: pl.BlockSpec(shape, lambda i: (i, 0))

    vec_out, mi = pl.pallas_call(
        _main_body,
        grid=(N // BR,),
        in_specs=[
            full((N, V * C)),
            blk((BR, V * C)),
            full((N, V)),
            blk((BR, V)),
            full((N, H)),
            blk((BR, H)),
            full((V, H)),
            full((H, H)),
            full((1, H)),
            full((H, H + 1)),
            full((1, H + 1)),
            full((H, H)),
            full((1, H)),
            full((H, V)),
            full((1, V)),
            full((V * C, V)),
            full((V, V * C)),
        ],
        out_specs=(
            blk((BR, V * C)),
            blk((BR, H)),
        ),
        out_shape=(
            jax.ShapeDtypeStruct((N, V * C), jnp.float32),
            jax.ShapeDtypeStruct((N, H), jnp.float32),
        ),
    )(xflat, xflat, n2, n2, fs, fr,
      We1[:V], We2, be2.reshape(1, H),
      jnp.concatenate([Wx1, Winf], axis=1),
      jnp.concatenate([bx1, binf]).reshape(1, H + 1),
      Wx2, bx2.reshape(1, H), Wxo, bxo.reshape(1, V),
      G, GT)

    features_out = pl.pallas_call(
        _epilogue_body,
        out_shape=jax.ShapeDtypeStruct((N, F), jnp.float32),
    )(mi, node_features, Wh1[:H], Wh1[H:], bh1.reshape(1, H), Wh2,
      bh2.reshape(1, H), Who, bho.reshape(1, F))

    return vec_out.reshape(N, V, C), features_out


# speedup vs baseline: 101.0685x; 1.3631x over previous
import jax
import jax.numpy as jnp
import numpy as np
from jax.experimental import pallas as pl

N = 512
V = 4
C = 3
F = 64
H = 64
BR = 8
N2 = N // 2
E2 = BR * N2
INV_NEIGH = 1.0 / (N - 1)
INV_SQRT_NEIGH = 1.0 / float(np.sqrt(N - 1))


def _silu(x):
    return x / (1.0 + jnp.exp(-x))


def _prologue_body(f_ref, xflat_ref, w1s_ref, w1r_ref, be1_ref, g_ref,
                   fs_ref, fr_ref, n2_ref):
    f = f_ref[...]
    x = xflat_ref[...]
    fs_ref[...] = jnp.dot(f, w1s_ref[...], preferred_element_type=jnp.float32) + be1_ref[...]
    fr_ref[...] = jnp.dot(f, w1r_ref[...], preferred_element_type=jnp.float32)
    n2_ref[...] = jnp.dot(x * x, g_ref[...], preferred_element_type=jnp.float32)


def _dup(a):
    return jnp.concatenate([a, a], axis=-1)


def _main_body(x2_ref, xr_ref, n2p_ref, n2r_ref, fs2_ref, fr_ref,
               w1v2_ref, we22_ref, be22_ref, wx12_ref, bx12_ref,
               wx22_ref, bx22_ref, wxo2_ref, bxo2_ref,
               winf2_ref, binf2_ref, g2_ref, gt2_ref, gt_ref, eb_ref,
               vec_out_ref, mi_out_ref):
    r0 = pl.program_id(0) * BR
    X2 = x2_ref[...]
    xr = xr_ref[...]

    XT = jnp.broadcast_to(X2[None], (BR, N2, 2 * V * C)).reshape(E2, 2 * V * C)
    xrT = jnp.broadcast_to(_dup(xr)[:, None, :], (BR, N2, 2 * V * C)).reshape(E2, 2 * V * C)

    cross = jnp.dot(XT * xrT, g2_ref[...], preferred_element_type=jnp.float32)
    n2sT = jnp.broadcast_to(n2p_ref[...][None], (BR, N2, 2 * V)).reshape(E2, 2 * V)
    n2rT = jnp.broadcast_to(_dup(n2r_ref[...])[:, None, :], (BR, N2, 2 * V)).reshape(E2, 2 * V)
    len2 = jnp.maximum(n2sT + n2rT - 2.0 * cross, 0.0)

    FsT = jnp.broadcast_to(fs2_ref[...][None], (BR, N2, 2 * H)).reshape(E2, 2 * H)
    FrT = jnp.broadcast_to(_dup(fr_ref[...])[:, None, :], (BR, N2, 2 * H)).reshape(E2, 2 * H)
    h = _silu(jnp.dot(len2, w1v2_ref[...], preferred_element_type=jnp.float32) + FsT + FrT)
    m = _silu(jnp.dot(h, we22_ref[...], preferred_element_type=jnp.float32) + be22_ref[...])

    px = _silu(jnp.dot(m, wx12_ref[...], preferred_element_type=jnp.float32) + bx12_ref[...])
    px = _silu(jnp.dot(px, wx22_ref[...], preferred_element_type=jnp.float32) + bx22_ref[...])
    po = jnp.dot(px, wxo2_ref[...], preferred_element_type=jnp.float32) + bxo2_ref[...]
    w = po / (1.0 + jnp.sqrt(len2))

    w24 = jnp.dot(w, gt2_ref[...], preferred_element_type=jnp.float32)
    c24 = (w24 * XT).reshape(BR, N2, 2 * V * C).sum(axis=1)
    contrib = c24[:, :V * C] + c24[:, V * C:]
    w8 = w.reshape(BR, N2, 2 * V).sum(axis=1)
    wsum = w8[:, :V] + w8[:, V:]
    shift = jnp.dot(wsum, gt_ref[...], preferred_element_type=jnp.float32) * xr - contrib
    vec_out_ref[...] = xr + shift * INV_NEIGH

    e2 = 1.0 / (1.0 + jnp.exp(-(jnp.dot(m, winf2_ref[...], preferred_element_type=jnp.float32)
                                + binf2_ref[...])))
    rowi = jax.lax.broadcasted_iota(jnp.int32, (E2, 1), 0)
    j2 = rowi & (N2 - 1)
    r = r0 + (rowi >> 8)
    mask = jnp.concatenate([(j2 * 2 != r).astype(jnp.float32),
                            (j2 * 2 + 1 != r).astype(jnp.float32)], axis=1)
    egate = jnp.dot(e2 * mask, eb_ref[...], preferred_element_type=jnp.float32)
    mi2 = (m * egate).reshape(BR, N2, 2 * H).sum(axis=1)
    mi_out_ref[...] = (mi2[:, :H] + mi2[:, H:]) * INV_SQRT_NEIGH


def _epilogue_body(mi_ref, f_ref, wh1a_ref, wh1b_ref, bh1_ref, wh2_ref,
                   bh2_ref, who_ref, bho_ref, out_ref):
    f = f_ref[...]
    ph = _silu(jnp.dot(mi_ref[...], wh1a_ref[...], preferred_element_type=jnp.float32)
               + jnp.dot(f, wh1b_ref[...], preferred_element_type=jnp.float32)
               + bh1_ref[...])
    ph = _silu(jnp.dot(ph, wh2_ref[...], preferred_element_type=jnp.float32) + bh2_ref[...])
    out_ref[...] = jnp.dot(ph, who_ref[...], preferred_element_type=jnp.float32) \
        + bho_ref[...] + f


def _group_sum_matrix():
    g = np.zeros((V * C, V), dtype=np.float32)
    for v in range(V):
        g[v * C:(v + 1) * C, v] = 1.0
    return jnp.asarray(g)


def _bdiag(a, b):
    za = jnp.zeros((a.shape[0], b.shape[1]), a.dtype)
    zb = jnp.zeros((b.shape[0], a.shape[1]), a.dtype)
    return jnp.concatenate(
        [jnp.concatenate([a, za], axis=1), jnp.concatenate([zb, b], axis=1)], axis=0)


@jax.jit
def kernel(node_vectors, node_features, We1, be1, We2, be2, Wx1, bx1, Wx2, bx2,
           Wxo, bxo, Winf, binf, Wh1, bh1, Wh2, bh2, Who, bho):
    xflat = node_vectors.reshape(N, V * C)
    G = _group_sum_matrix()
    GT = G.T
    W1v = We1[:V]

    fs, fr, n2 = pl.pallas_call(
        _prologue_body,
        out_shape=(
            jax.ShapeDtypeStruct((N, H), jnp.float32),
            jax.ShapeDtypeStruct((N, H), jnp.float32),
            jax.ShapeDtypeStruct((N, V), jnp.float32),
        ),
    )(node_features, xflat, We1[V:V + F], We1[V + F:], be1.reshape(1, H), G)

    eb = np.zeros((2, 2 * H), dtype=np.float32)
    eb[0, :H] = 1.0
    eb[1, H:] = 1.0

    full = lambda shape: pl.BlockSpec(shape, lambda i: (0, 0))
    blk = lambda shape: pl.BlockSpec(shape, lambda i: (i, 0))

    vec_out, mi = pl.pallas_call(
        _main_body,
        grid=(N // BR,),
        in_specs=[
            full((N2, 2 * V * C)),
            blk((BR, V * C)),
            full((N2, 2 * V)),
            blk((BR, V)),
            full((N2, 2 * H)),
            blk((BR, H)),
            full((2 * V, 2 * H)),
            full((2 * H, 2 * H)),
            full((1, 2 * H)),
            full((2 * H, 2 * H)),
            full((1, 2 * H)),
            full((2 * H, 2 * H)),
            full((1, 2 * H)),
            full((2 * H, 2 * V)),
            full((1, 2 * V)),
            full((2 * H, 2)),
            full((1, 2)),
            full((2 * V * C, 2 * V)),
            full((2 * V, 2 * V * C)),
            full((V, V * C)),
            full((2, 2 * H)),
        ],
        out_specs=(
            blk((BR, V * C)),
            blk((BR, H)),
        ),
        out_shape=(
            jax.ShapeDtypeStruct((N, V * C), jnp.float32),
            jax.ShapeDtypeStruct((N, H), jnp.float32),
        ),
    )(xflat.reshape(N2, 2 * V * C), xflat, n2.reshape(N2, 2 * V), n2,
      fs.reshape(N2, 2 * H), fr,
      _bdiag(W1v, W1v), _bdiag(We2, We2), jnp.tile(be2, 2).reshape(1, 2 * H),
      _bdiag(Wx1, Wx1), jnp.tile(bx1, 2).reshape(1, 2 * H),
      _bdiag(Wx2, Wx2), jnp.tile(bx2, 2).reshape(1, 2 * H),
      _bdiag(Wxo, Wxo), jnp.tile(bxo, 2).reshape(1, 2 * V),
      _bdiag(Winf, Winf), jnp.tile(binf, 2).reshape(1, 2),
      _bdiag(G, G), _bdiag(GT, GT), GT, jnp.asarray(eb))

    features_out = pl.pallas_call(
        _epilogue_body,
        out_shape=jax.ShapeDtypeStruct((N, F), jnp.float32),
    )(mi, node_features, Wh1[:H], Wh1[H:], bh1.reshape(1, H), Wh2,
      bh2.reshape(1, H), Who, bho.reshape(1, F))

    return vec_out.reshape(N, V, C), features_out


# speedup vs baseline: 124.2554x; 1.2294x over previous
import jax
import jax.numpy as jnp
import numpy as np
from jax.experimental import pallas as pl

N = 512
V = 4
C = 3
F = 64
H = 64
BR = 8
N2 = N // 2
E2 = BR * N2
INV_NEIGH = 1.0 / (N - 1)
INV_SQRT_NEIGH = 1.0 / float(np.sqrt(N - 1))


def _sigmoid(x):
    return 0.5 + 0.5 * jnp.tanh(0.5 * x)


def _silu(x):
    t = 0.5 * x
    return t + t * jnp.tanh(t)


def _prologue_body(f_ref, xflat_ref, w1s_ref, w1r_ref, be1_ref, g_ref,
                   fs_ref, fr_ref, n2_ref):
    f = f_ref[...]
    x = xflat_ref[...]
    fs_ref[...] = jnp.dot(f, w1s_ref[...], preferred_element_type=jnp.float32) + be1_ref[...]
    fr_ref[...] = jnp.dot(f, w1r_ref[...], preferred_element_type=jnp.float32)
    n2_ref[...] = jnp.dot(x * x, g_ref[...], preferred_element_type=jnp.float32)


def _dup(a):
    return jnp.concatenate([a, a], axis=-1)


def _main_body(x2_ref, xr_ref, n2p_ref, n2r_ref, fs2_ref, fr_ref,
               w1v2_ref, we22_ref, be22_ref, wx12_ref, bx12_ref,
               wx22_ref, bx22_ref, wxo2_ref, bxo2_ref,
               winf2_ref, binf2_ref, g2_ref, gt2_ref, gt_ref, eb_ref,
               vec_out_ref, mi_out_ref):
    X2 = x2_ref[...]
    xr = xr_ref[...]

    XT = jnp.broadcast_to(X2[None], (BR, N2, 2 * V * C)).reshape(E2, 2 * V * C)
    xrT = jnp.broadcast_to(_dup(xr)[:, None, :], (BR, N2, 2 * V * C)).reshape(E2, 2 * V * C)

    cross = jnp.dot(XT * xrT, g2_ref[...], preferred_element_type=jnp.float32)
    n2sT = jnp.broadcast_to(n2p_ref[...][None], (BR, N2, 2 * V)).reshape(E2, 2 * V)
    n2rT = jnp.broadcast_to(_dup(n2r_ref[...])[:, None, :], (BR, N2, 2 * V)).reshape(E2, 2 * V)
    len2 = jnp.maximum(n2sT + n2rT - 2.0 * cross, 0.0)

    FsT = jnp.broadcast_to(fs2_ref[...][None], (BR, N2, 2 * H)).reshape(E2, 2 * H)
    FrT = jnp.broadcast_to(_dup(fr_ref[...])[:, None, :], (BR, N2, 2 * H)).reshape(E2, 2 * H)
    h = _silu(jnp.dot(len2, w1v2_ref[...], preferred_element_type=jnp.float32) + FsT + FrT)
    m = _silu(jnp.dot(h, we22_ref[...], preferred_element_type=jnp.float32) + be22_ref[...])

    px = _silu(jnp.dot(m, wx12_ref[...], preferred_element_type=jnp.float32) + bx12_ref[...])
    px = _silu(jnp.dot(px, wx22_ref[...], preferred_element_type=jnp.float32) + bx22_ref[...])
    po = jnp.dot(px, wxo2_ref[...], preferred_element_type=jnp.float32) + bxo2_ref[...]
    w = po / (1.0 + jnp.sqrt(len2))

    w24 = jnp.dot(w, gt2_ref[...], preferred_element_type=jnp.float32)
    c24 = (w24 * XT).reshape(BR, N2, 2 * V * C).sum(axis=1)
    contrib = c24[:, :V * C] + c24[:, V * C:]
    w8 = w.reshape(BR, N2, 2 * V).sum(axis=1)
    wsum = w8[:, :V] + w8[:, V:]
    shift = jnp.dot(wsum, gt_ref[...], preferred_element_type=jnp.float32) * xr - contrib
    vec_out_ref[...] = xr + shift * INV_NEIGH

    e2 = _sigmoid(jnp.dot(m, winf2_ref[...], preferred_element_type=jnp.float32)
                  + binf2_ref[...])
    egate = jnp.dot(e2, eb_ref[...], preferred_element_type=jnp.float32)
    mi2 = (m * egate).reshape(BR, N2, 2 * H).sum(axis=1)
    mi_out_ref[...] = (mi2[:, :H] + mi2[:, H:]) * INV_SQRT_NEIGH


def _epilogue_body(mi_ref, f_ref, fs_ref, fr_ref, we2_ref, be2_ref,
                   winf_ref, binf_ref, wh1a_ref, wh1b_ref, bh1_ref, wh2_ref,
                   bh2_ref, who_ref, bho_ref, out_ref):
    f = f_ref[...]
    md = _silu(fs_ref[...] + fr_ref[...])
    md = _silu(jnp.dot(md, we2_ref[...], preferred_element_type=jnp.float32) + be2_ref[...])
    ed = _sigmoid(jnp.dot(md, winf_ref[...], preferred_element_type=jnp.float32)
                  + binf_ref[...])
    mi = mi_ref[...] - md * ed * INV_SQRT_NEIGH
    ph = _silu(jnp.dot(mi, wh1a_ref[...], preferred_element_type=jnp.float32)
               + jnp.dot(f, wh1b_ref[...], preferred_element_type=jnp.float32)
               + bh1_ref[...])
    ph = _silu(jnp.dot(ph, wh2_ref[...], preferred_element_type=jnp.float32) + bh2_ref[...])
    out_ref[...] = jnp.dot(ph, who_ref[...], preferred_element_type=jnp.float32) \
        + bho_ref[...] + f


def _group_sum_matrix():
    g = np.zeros((V * C, V), dtype=np.float32)
    for v in range(V):
        g[v * C:(v + 1) * C, v] = 1.0
    return jnp.asarray(g)


def _bdiag(a, b):
    za = jnp.zeros((a.shape[0], b.shape[1]), a.dtype)
    zb = jnp.zeros((b.shape[0], a.shape[1]), a.dtype)
    return jnp.concatenate(
        [jnp.concatenate([a, za], axis=1), jnp.concatenate([zb, b], axis=1)], axis=0)


@jax.jit
def kernel(node_vectors, node_features, We1, be1, We2, be2, Wx1, bx1, Wx2, bx2,
           Wxo, bxo, Winf, binf, Wh1, bh1, Wh2, bh2, Who, bho):
    xflat = node_vectors.reshape(N, V * C)
    G = _group_sum_matrix()
    GT = G.T
    W1v = We1[:V]

    fs, fr, n2 = pl.pallas_call(
        _prologue_body,
        out_shape=(
            jax.ShapeDtypeStruct((N, H), jnp.float32),
            jax.ShapeDtypeStruct((N, H), jnp.float32),
            jax.ShapeDtypeStruct((N, V), jnp.float32),
        ),
    )(node_features, xflat, We1[V:V + F], We1[V + F:], be1.reshape(1, H), G)

    eb = np.zeros((2, 2 * H), dtype=np.float32)
    eb[0, :H] = 1.0
    eb[1, H:] = 1.0

    full = lambda shape: pl.BlockSpec(shape, lambda i: (0, 0))
    blk = lambda shape: pl.BlockSpec(shape, lambda i: (i, 0))

    vec_out, mi = pl.pallas_call(
        _main_body,
        grid=(N // BR,),
        in_specs=[
            full((N2, 2 * V * C)),
            blk((BR, V * C)),
            full((N2, 2 * V)),
            blk((BR, V)),
            full((N2, 2 * H)),
            blk((BR, H)),
            full((2 * V, 2 * H)),
            full((2 * H, 2 * H)),
            full((1, 2 * H)),
            full((2 * H, 2 * H)),
            full((1, 2 * H)),
            full((2 * H, 2 * H)),
            full((1, 2 * H)),
            full((2 * H, 2 * V)),
            full((1, 2 * V)),
            full((2 * H, 2)),
            full((1, 2)),
            full((2 * V * C, 2 * V)),
            full((2 * V, 2 * V * C)),
            full((V, V * C)),
            full((2, 2 * H)),
        ],
        out_specs=(
            blk((BR, V * C)),
            blk((BR, H)),
        ),
        out_shape=(
            jax.ShapeDtypeStruct((N, V * C), jnp.float32),
            jax.ShapeDtypeStruct((N, H), jnp.float32),
        ),
    )(xflat.reshape(N2, 2 * V * C), xflat, n2.reshape(N2, 2 * V), n2,
      fs.reshape(N2, 2 * H), fr,
      _bdiag(W1v, W1v), _bdiag(We2, We2), jnp.tile(be2, 2).reshape(1, 2 * H),
      _bdiag(Wx1, Wx1), jnp.tile(bx1, 2).reshape(1, 2 * H),
      _bdiag(Wx2, Wx2), jnp.tile(bx2, 2).reshape(1, 2 * H),
      _bdiag(Wxo, Wxo), jnp.tile(bxo, 2).reshape(1, 2 * V),
      _bdiag(Winf, Winf), jnp.tile(binf, 2).reshape(1, 2),
      _bdiag(G, G), _bdiag(GT, GT), GT, jnp.asarray(eb))

    features_out = pl.pallas_call(
        _epilogue_body,
        out_shape=jax.ShapeDtypeStruct((N, F), jnp.float32),
    )(mi, node_features, fs, fr, We2, be2.reshape(1, H),
      Winf, binf.reshape(1, 1),
      Wh1[:H], Wh1[H:], bh1.reshape(1, H), Wh2,
      bh2.reshape(1, H), Who, bho.reshape(1, F))

    return vec_out.reshape(N, V, C), features_out


# speedup vs baseline: 154.9028x; 1.2466x over previous
import jax
import jax.numpy as jnp
import numpy as np
from jax.experimental import pallas as pl

N = 512
V = 4
C = 3
F = 64
H = 64
BR = 16
N2 = N // 2
E2 = BR * N2
D = V * C
INV_NEIGH = 1.0 / (N - 1)
INV_SQRT_NEIGH = 1.0 / float(np.sqrt(N - 1))


def _sigmoid(x):
    return 0.5 + 0.5 * jnp.tanh(0.5 * x)


def _silu(x):
    t = 0.5 * x
    return t + t * jnp.tanh(t)


def _prologue_body(f_ref, xflat_ref, w1s_ref, w1r_ref, be1_ref, gg_ref,
                   fs_ref, fr_ref, n24_ref):
    f = f_ref[...]
    x = xflat_ref[...]
    fs_ref[...] = jnp.dot(f, w1s_ref[...], preferred_element_type=jnp.float32) + be1_ref[...]
    fr_ref[...] = jnp.dot(f, w1r_ref[...], preferred_element_type=jnp.float32)
    n24_ref[...] = jnp.dot(x * x, gg_ref[...], preferred_element_type=jnp.float32)


def _dup(a):
    return jnp.concatenate([a, a], axis=-1)


def _main_body(x2_ref, xr_ref, n24p_ref, n24r_ref, fs2_ref, fr_ref,
               w1v24_ref, we22_ref, be22_ref, wx12_ref, bx12_ref,
               wx22_ref, bx22_ref, wxo24_ref, bxo24_ref,
               winfw_ref, binfw_ref, g24_ref,
               vec_out_ref, mi_out_ref):
    X2 = x2_ref[...]
    xr = xr_ref[...]

    XT = jnp.broadcast_to(X2[None], (BR, N2, 2 * D)).reshape(E2, 2 * D)
    xrT = jnp.broadcast_to(_dup(xr)[:, None, :], (BR, N2, 2 * D)).reshape(E2, 2 * D)

    cross = jnp.dot(XT * xrT, g24_ref[...], preferred_element_type=jnp.float32)
    n2sT = jnp.broadcast_to(n24p_ref[...][None], (BR, N2, 2 * D)).reshape(E2, 2 * D)
    n2rT = jnp.broadcast_to(_dup(n24r_ref[...])[:, None, :], (BR, N2, 2 * D)).reshape(E2, 2 * D)
    len2 = jnp.maximum(n2sT + n2rT - 2.0 * cross, 0.0)
    invden = 1.0 / (1.0 + len2 * jax.lax.rsqrt(len2 + 1e-30))

    FsT = jnp.broadcast_to(fs2_ref[...][None], (BR, N2, 2 * H)).reshape(E2, 2 * H)
    FrT = jnp.broadcast_to(_dup(fr_ref[...])[:, None, :], (BR, N2, 2 * H)).reshape(E2, 2 * H)
    h = _silu(jnp.dot(len2, w1v24_ref[...], preferred_element_type=jnp.float32) + FsT + FrT)
    m = _silu(jnp.dot(h, we22_ref[...], preferred_element_type=jnp.float32) + be22_ref[...])

    px = _silu(jnp.dot(m, wx12_ref[...], preferred_element_type=jnp.float32) + bx12_ref[...])
    px = _silu(jnp.dot(px, wx22_ref[...], preferred_element_type=jnp.float32) + bx22_ref[...])
    po = jnp.dot(px, wxo24_ref[...], preferred_element_type=jnp.float32) + bxo24_ref[...]
    w24 = po * invden

    c24 = (w24 * XT).reshape(BR, N2, 2 * D).sum(axis=1)
    contrib = c24[:, :D] + c24[:, D:]
    s24 = w24.reshape(BR, N2, 2 * D).sum(axis=1)
    wsum = s24[:, :D] + s24[:, D:]
    vec_out_ref[...] = xr + (wsum * xr - contrib) * INV_NEIGH

    egate = _sigmoid(jnp.dot(m, winfw_ref[...], preferred_element_type=jnp.float32)
                     + binfw_ref[...])
    mi2 = (m * egate).reshape(BR, N2, 2 * H).sum(axis=1)
    mi_out_ref[...] = (mi2[:, :H] + mi2[:, H:]) * INV_SQRT_NEIGH


def _epilogue_body(mi_ref, f_ref, fs_ref, fr_ref, we2_ref, be2_ref,
                   winf_ref, binf_ref, wh1a_ref, wh1b_ref, bh1_ref, wh2_ref,
                   bh2_ref, who_ref, bho_ref, out_ref):
    f = f_ref[...]
    md = _silu(fs_ref[...] + fr_ref[...])
    md = _silu(jnp.dot(md, we2_ref[...], preferred_element_type=jnp.float32) + be2_ref[...])
    ed = _sigmoid(jnp.dot(md, winf_ref[...], preferred_element_type=jnp.float32)
                  + binf_ref[...])
    mi = mi_ref[...] - md * ed * INV_SQRT_NEIGH
    ph = _silu(jnp.dot(mi, wh1a_ref[...], preferred_element_type=jnp.float32)
               + jnp.dot(f, wh1b_ref[...], preferred_element_type=jnp.float32)
               + bh1_ref[...])
    ph = _silu(jnp.dot(ph, wh2_ref[...], preferred_element_type=jnp.float32) + bh2_ref[...])
    out_ref[...] = jnp.dot(ph, who_ref[...], preferred_element_type=jnp.float32) \
        + bho_ref[...] + f


def _group_sum_matrix():
    g = np.zeros((D, V), dtype=np.float32)
    for v in range(V):
        g[v * C:(v + 1) * C, v] = 1.0
    return jnp.asarray(g)


def _bdiag(a, b):
    za = jnp.zeros((a.shape[0], b.shape[1]), a.dtype)
    zb = jnp.zeros((b.shape[0], a.shape[1]), a.dtype)
    return jnp.concatenate(
        [jnp.concatenate([a, za], axis=1), jnp.concatenate([zb, b], axis=1)], axis=0)


@jax.jit
def kernel(node_vectors, node_features, We1, be1, We2, be2, Wx1, bx1, Wx2, bx2,
           Wxo, bxo, Winf, binf, Wh1, bh1, Wh2, bh2, Who, bho):
    xflat = node_vectors.reshape(N, D)
    G = _group_sum_matrix()
    GT = G.T
    GG = G @ GT

    fs, fr, n24 = pl.pallas_call(
        _prologue_body,
        out_shape=(
            jax.ShapeDtypeStruct((N, H), jnp.float32),
            jax.ShapeDtypeStruct((N, H), jnp.float32),
            jax.ShapeDtypeStruct((N, D), jnp.float32),
        ),
    )(node_features, xflat, We1[V:V + F], We1[V + F:], be1.reshape(1, H), GG)

    eb = np.zeros((2, 2 * H), dtype=np.float32)
    eb[0, :H] = 1.0
    eb[1, H:] = 1.0
    eb = jnp.asarray(eb)
    w1v12 = (G @ We1[:V]) / C
    wxo12 = Wxo @ GT
    winfw = _bdiag(Winf, Winf) @ eb
    binfw = jnp.tile(binf, 2).reshape(1, 2) @ eb

    full = lambda shape: pl.BlockSpec(shape, lambda i: (0, 0))
    blk = lambda shape: pl.BlockSpec(shape, lambda i: (i, 0))

    vec_out, mi = pl.pallas_call(
        _main_body,
        grid=(N // BR,),
        in_specs=[
            full((N2, 2 * D)),
            blk((BR, D)),
            full((N2, 2 * D)),
            blk((BR, D)),
            full((N2, 2 * H)),
            blk((BR, H)),
            full((2 * D, 2 * H)),
            full((2 * H, 2 * H)),
            full((1, 2 * H)),
            full((2 * H, 2 * H)),
            full((1, 2 * H)),
            full((2 * H, 2 * H)),
            full((1, 2 * H)),
            full((2 * H, 2 * D)),
            full((1, 2 * D)),
            full((2 * H, 2 * H)),
            full((1, 2 * H)),
            full((2 * D, 2 * D)),
        ],
        out_specs=(
            blk((BR, D)),
            blk((BR, H)),
        ),
        out_shape=(
            jax.ShapeDtypeStruct((N, D), jnp.float32),
            jax.ShapeDtypeStruct((N, H), jnp.float32),
        ),
    )(xflat.reshape(N2, 2 * D), xflat, n24.reshape(N2, 2 * D), n24,
      fs.reshape(N2, 2 * H), fr,
      _bdiag(w1v12, w1v12), _bdiag(We2, We2), jnp.tile(be2, 2).reshape(1, 2 * H),
      _bdiag(Wx1, Wx1), jnp.tile(bx1, 2).reshape(1, 2 * H),
      _bdiag(Wx2, Wx2), jnp.tile(bx2, 2).reshape(1, 2 * H),
      _bdiag(wxo12, wxo12), jnp.tile(bxo.reshape(1, V) @ GT, (1, 2)),
      winfw, binfw, _bdiag(GG, GG))

    features_out = pl.pallas_call(
        _epilogue_body,
        out_shape=jax.ShapeDtypeStruct((N, F), jnp.float32),
    )(mi, node_features, fs, fr, We2, be2.reshape(1, H),
      Winf, binf.reshape(1, 1),
      Wh1[:H], Wh1[H:], bh1.reshape(1, H), Wh2,
      bh2.reshape(1, H), Who, bho.reshape(1, F))

    return vec_out.reshape(N, V, C), features_out


# speedup vs baseline: 171.9936x; 1.1103x over previous
import jax
import jax.numpy as jnp
import numpy as np
from jax.experimental import pallas as pl

N = 512
V = 4
C = 3
F = 64
H = 64
BR = 16
N2 = N // 2
E2 = BR * N2
D = V * C
A = 2 * D + D + 2 * D + BR
L = 2 * D + BR
INV_NEIGH = 1.0 / (N - 1)
INV_SQRT_NEIGH = 1.0 / float(np.sqrt(N - 1))


def _silu_h(t):
    return t + t * jnp.tanh(t)


def _sigmoid_h(t):
    return 0.5 + 0.5 * jnp.tanh(t)


def _prologue_body(f_ref, xflat_ref, w1s_ref, w1r_ref, be1_ref, gg_ref,
                   fs_ref, fr_ref, n24_ref):
    f = f_ref[...]
    x = xflat_ref[...]
    fs_ref[...] = jnp.dot(f, w1s_ref[...], preferred_element_type=jnp.float32) + be1_ref[...]
    fr_ref[...] = jnp.dot(f, w1r_ref[...], preferred_element_type=jnp.float32)
    n24_ref[...] = jnp.dot(x * x, gg_ref[...], preferred_element_type=jnp.float32)


def _dup(a):
    return jnp.concatenate([a, a], axis=-1)


def _main_body(xta_ref, fst_ref,
               xr_ref, n24r_ref, fr_ref, eye_ref,
               w1v24_ref, we22_ref, be22_ref, wx12_ref, bx12_ref,
               wx22_ref, bx22_ref, wxo24_ref, bxo24_ref,
               winfw_ref, binfw_ref, gaug_ref,
               vec_out_ref, mi_out_ref):
    XTA = xta_ref[...]
    xr = xr_ref[...]

    xra = jnp.concatenate(
        [_dup(xr), n24r_ref[...], jnp.ones((BR, 2 * D), jnp.float32), eye_ref[...]],
        axis=1)
    xrT = jnp.broadcast_to(xra[:, None, :], (BR, N2, A)).reshape(E2, A)

    la = jnp.dot(XTA * xrT, gaug_ref[...], preferred_element_type=jnp.float32)
    la = jnp.maximum(la, 1e-30)
    invden = 1.0 / (1.0 + la * jax.lax.rsqrt(la))

    w1a = jnp.concatenate([w1v24_ref[...], _dup(fr_ref[...])], axis=0)
    h = _silu_h(jnp.dot(la, w1a, preferred_element_type=jnp.float32) + fst_ref[...])
    m = _silu_h(jnp.dot(h, we22_ref[...], preferred_element_type=jnp.float32) + be22_ref[...])

    px = _silu_h(jnp.dot(m, wx12_ref[...], preferred_element_type=jnp.float32) + bx12_ref[...])
    px = _silu_h(jnp.dot(px, wx22_ref[...], preferred_element_type=jnp.float32) + bx22_ref[...])
    po = jnp.dot(px, wxo24_ref[...], preferred_element_type=jnp.float32) + bxo24_ref[...]
    w24 = po * invden[:, :2 * D]

    c24 = (w24 * XTA[:, :2 * D]).reshape(BR, N2, 2 * D).sum(axis=1)
    s24 = w24.reshape(BR, N2, 2 * D).sum(axis=1)
    contrib = c24[:, :D] + c24[:, D:]
    wsum = s24[:, :D] + s24[:, D:]
    vec_out_ref[...] = xr + (wsum * xr - contrib) * INV_NEIGH

    tg = jnp.dot(m, winfw_ref[...], preferred_element_type=jnp.float32) + binfw_ref[...]
    mg = m + m * jnp.tanh(tg)
    mi2 = mg.reshape(BR, N2, 2 * H).sum(axis=1)
    mi_out_ref[...] = (mi2[:, :H] + mi2[:, H:]) * (0.5 * INV_SQRT_NEIGH)


def _epilogue_body(mi_ref, f_ref, fs_ref, fr_ref, we2_ref, be2_ref,
                   winf_ref, binf_ref, wh1a_ref, wh1b_ref, bh1_ref, wh2_ref,
                   bh2_ref, who_ref, bho_ref, out_ref):
    f = f_ref[...]
    md = _silu_h(fs_ref[...] + fr_ref[...])
    md = _silu_h(jnp.dot(md, we2_ref[...], preferred_element_type=jnp.float32) + be2_ref[...])
    ed = _sigmoid_h(jnp.dot(md, winf_ref[...], preferred_element_type=jnp.float32)
                    + binf_ref[...])
    mi = mi_ref[...] - md * ed * INV_SQRT_NEIGH
    ph = _silu_h(jnp.dot(mi, wh1a_ref[...], preferred_element_type=jnp.float32)
                 + jnp.dot(f, wh1b_ref[...], preferred_element_type=jnp.float32)
                 + bh1_ref[...])
    ph = _silu_h(jnp.dot(ph, wh2_ref[...], preferred_element_type=jnp.float32) + bh2_ref[...])
    out_ref[...] = jnp.dot(ph, who_ref[...], preferred_element_type=jnp.float32) \
        + bho_ref[...] + f


def _group_sum_matrix():
    g = np.zeros((D, V), dtype=np.float32)
    for v in range(V):
        g[v * C:(v + 1) * C, v] = 1.0
    return g


def _bdiag(a, b):
    za = jnp.zeros((a.shape[0], b.shape[1]), jnp.float32)
    zb = jnp.zeros((b.shape[0], a.shape[1]), jnp.float32)
    return jnp.concatenate(
        [jnp.concatenate([a, za], axis=1), jnp.concatenate([zb, b], axis=1)], axis=0)


_G_NP = _group_sum_matrix()
_GG_NP = _G_NP @ _G_NP.T


def _gaug_np():
    g = np.zeros((A, L), dtype=np.float32)
    g[0:D, 0:D] = -2.0 * _GG_NP
    g[D:2 * D, D:2 * D] = -2.0 * _GG_NP
    g[2 * D:3 * D, 0:D] = np.eye(D)
    g[2 * D:3 * D, D:2 * D] = np.eye(D)
    g[3 * D:5 * D, 0:2 * D] = np.eye(2 * D)
    g[5 * D:A, 2 * D:L] = np.eye(BR)
    return g


_GAUG_NP = _gaug_np()
_EYE_NP = np.eye(BR, dtype=np.float32)


@jax.jit
def kernel(node_vectors, node_features, We1, be1, We2, be2, Wx1, bx1, Wx2, bx2,
           Wxo, bxo, Winf, binf, Wh1, bh1, Wh2, bh2, Who, bho):
    xflat = node_vectors.reshape(N, D)
    G = jnp.asarray(_G_NP)
    GT = G.T
    GG = jnp.asarray(_GG_NP)

    fs, fr, n24 = pl.pallas_call(
        _prologue_body,
        out_shape=(
            jax.ShapeDtypeStruct((N, H), jnp.float32),
            jax.ShapeDtypeStruct((N, H), jnp.float32),
            jax.ShapeDtypeStruct((N, D), jnp.float32),
        ),
    )(node_features, xflat, 0.5 * We1[V:V + F], 0.5 * We1[V + F:],
      0.5 * be1.reshape(1, H), GG)

    x2 = xflat.reshape(N2, 2 * D)
    xta_base = jnp.concatenate(
        [x2, jnp.ones((N2, D), jnp.float32), n24.reshape(N2, 2 * D),
         jnp.ones((N2, BR), jnp.float32)], axis=1)
    xta_full = jnp.tile(xta_base, (BR, 1))
    fst_full = jnp.tile(fs.reshape(N2, 2 * H), (BR, 1))

    eb = np.zeros((2, 2 * H), dtype=np.float32)
    eb[0, :H] = 1.0
    eb[1, H:] = 1.0
    eb = jnp.asarray(eb)
    w1v12 = (G @ We1[:V]) / C
    wxo12 = Wxo @ GT
    winfw = _bdiag(Winf, Winf) @ eb
    binfw = jnp.tile(binf, 2).reshape(1, 2) @ eb

    full = lambda shape: pl.BlockSpec(shape, lambda i: (0, 0))
    blk = lambda shape: pl.BlockSpec(shape, lambda i: (i, 0))

    vec_out, mi = pl.pallas_call(
        _main_body,
        grid=(N // BR,),
        in_specs=[
            full((E2, A)),
            full((E2, 2 * H)),
            blk((BR, D)),
            blk((BR, D)),
            blk((BR, H)),
            full((BR, BR)),
            full((2 * D, 2 * H)),
            full((2 * H, 2 * H)),
            full((1, 2 * H)),
            full((2 * H, 2 * H)),
            full((1, 2 * H)),
            full((2 * H, 2 * H)),
            full((1, 2 * H)),
            full((2 * H, 2 * D)),
            full((1, 2 * D)),
            full((2 * H, 2 * H)),
            full((1, 2 * H)),
            full((A, L)),
        ],
        out_specs=(
            blk((BR, D)),
            blk((BR, H)),
        ),
        out_shape=(
            jax.ShapeDtypeStruct((N, D), jnp.float32),
            jax.ShapeDtypeStruct((N, H), jnp.float32),
        ),
    )(xta_full, fst_full,
      xflat, n24, fr, jnp.asarray(_EYE_NP),
      0.5 * _bdiag(w1v12, w1v12), 0.5 * _bdiag(We2, We2),
      0.5 * jnp.tile(be2, 2).reshape(1, 2 * H),
      0.5 * _bdiag(Wx1, Wx1), 0.5 * jnp.tile(bx1, 2).reshape(1, 2 * H),
      0.5 * _bdiag(Wx2, Wx2), 0.5 * jnp.tile(bx2, 2).reshape(1, 2 * H),
      _bdiag(wxo12, wxo12), jnp.tile(bxo.reshape(1, V) @ GT, (1, 2)),
      0.5 * winfw, 0.5 * binfw, jnp.asarray(_GAUG_NP))

    features_out = pl.pallas_call(
        _epilogue_body,
        out_shape=jax.ShapeDtypeStruct((N, F), jnp.float32),
    )(mi, node_features, fs, fr, 0.5 * We2, 0.5 * be2.reshape(1, H),
      0.5 * Winf, 0.5 * binf.reshape(1, 1),
      0.5 * Wh1[:H], 0.5 * Wh1[H:], 0.5 * bh1.reshape(1, H), 0.5 * Wh2,
      0.5 * bh2.reshape(1, H), Who, bho.reshape(1, F))

    return vec_out.reshape(N, V, C), features_out


# speedup vs baseline: 178.9430x; 1.0404x over previous
import jax
import jax.numpy as jnp
import numpy as np
from jax.experimental import pallas as pl
from jax.experimental.pallas import tpu as pltpu

N = 512
V = 4
C = 3
F = 64
H = 64
BR = 16
NB = N // BR
N2 = N // 2
E2 = BR * N2
D = V * C
A = 2 * D + D + 2 * D + BR
L = 2 * D + BR
INV_NEIGH = 1.0 / (N - 1)
INV_SQRT_NEIGH = 1.0 / float(np.sqrt(N - 1))


def _silu_h(t):
    return t + t * jnp.tanh(t)


def _sigmoid_h(t):
    return 0.5 + 0.5 * jnp.tanh(t)


def _dup(a):
    return jnp.concatenate([a, a], axis=-1)


def _body(f_ref, f2_ref, xflat_ref, x2_ref, xr_ref, eye_ref,
          w1s_ref, w1s2_ref, w1r_ref, be1_ref, gg_ref, gg2_ref,
          w1v24_ref, we22_ref, be22_ref, wx12_ref, bx12_ref,
          wx22_ref, bx22_ref, wxo24_ref, bxo24_ref,
          winfw_ref, binfw_ref, gaug_ref,
          we2h_ref, be2h_ref, winfh_ref, binfh_ref,
          wh1a_ref, wh1b_ref, bh1_ref, wh2_ref, bh2_ref, who_ref, bho_ref,
          vec_out_ref, fo_ref,
          fs_s, fr_s, n24_s, xta_s, fst_s, mi_s):
    i = pl.program_id(0)

    @pl.when(i == 0)
    def _prologue():
        f = f_ref[...]
        x = xflat_ref[...]
        x2 = x2_ref[...]
        fs_s[...] = jnp.dot(f, w1s_ref[...], preferred_element_type=jnp.float32) + be1_ref[...]
        fr_s[...] = jnp.dot(f, w1r_ref[...], preferred_element_type=jnp.float32)
        n24_s[...] = jnp.dot(x * x, gg_ref[...], preferred_element_type=jnp.float32)
        n24p = jnp.dot(x2 * x2, gg2_ref[...], preferred_element_type=jnp.float32)
        base = jnp.concatenate(
            [x2, jnp.ones((N2, D), jnp.float32), n24p,
             jnp.ones((N2, BR), jnp.float32)], axis=1)
        xta_s[...] = jnp.broadcast_to(base[None], (BR, N2, A)).reshape(E2, A)
        fs2 = jnp.dot(f2_ref[...], w1s2_ref[...],
                      preferred_element_type=jnp.float32) + _dup(be1_ref[...])
        fst_s[...] = jnp.broadcast_to(fs2[None], (BR, N2, 2 * H)).reshape(E2, 2 * H)

    XTA = xta_s[...]
    xr = xr_ref[...]
    n24r = n24_s[pl.ds(i * BR, BR), :]
    fr_blk = fr_s[pl.ds(i * BR, BR), :]

    xra = jnp.concatenate(
        [_dup(xr), n24r, jnp.ones((BR, 2 * D), jnp.float32), eye_ref[...]],
        axis=1)
    prod = (XTA.reshape(BR, N2, A) * xra[:, None, :]).reshape(E2, A)

    la = jnp.dot(prod, gaug_ref[...], preferred_element_type=jnp.float32)
    la = jnp.maximum(la, 1e-30)
    invden = 1.0 / (1.0 + la * jax.lax.rsqrt(la))

    w1a = jnp.concatenate([w1v24_ref[...], _dup(fr_blk)], axis=0)
    h = _silu_h(jnp.dot(la, w1a, preferred_element_type=jnp.float32) + fst_s[...])
    m = _silu_h(jnp.dot(h, we22_ref[...], preferred_element_type=jnp.float32) + be22_ref[...])

    px = _silu_h(jnp.dot(m, wx12_ref[...], preferred_element_type=jnp.float32) + bx12_ref[...])
    px = _silu_h(jnp.dot(px, wx22_ref[...], preferred_element_type=jnp.float32) + bx22_ref[...])
    po = jnp.dot(px, wxo24_ref[...], preferred_element_type=jnp.float32) + bxo24_ref[...]
    w24 = po * invden[:, :2 * D]

    c24 = (w24 * XTA[:, :2 * D]).reshape(BR, N2, 2 * D).sum(axis=1)
    s24 = w24.reshape(BR, N2, 2 * D).sum(axis=1)
    contrib = c24[:, :D] + c24[:, D:]
    wsum = s24[:, :D] + s24[:, D:]
    vec_out_ref[...] = xr + (wsum * xr - contrib) * INV_NEIGH

    tg = jnp.dot(m, winfw_ref[...], preferred_element_type=jnp.float32) + binfw_ref[...]
    mg = m + m * jnp.tanh(tg)
    mi2 = mg.reshape(BR, N2, 2 * H).sum(axis=1)
    mi_s[pl.ds(i * BR, BR), :] = (mi2[:, :H] + mi2[:, H:]) * (0.5 * INV_SQRT_NEIGH)

    @pl.when(i == NB - 1)
    def _epilogue():
        f = f_ref[...]
        md = _silu_h(fs_s[...] + fr_s[...])
        md = _silu_h(jnp.dot(md, we2h_ref[...], preferred_element_type=jnp.float32)
                     + be2h_ref[...])
        ed = _sigmoid_h(jnp.dot(md, winfh_ref[...], preferred_element_type=jnp.float32)
                        + binfh_ref[...])
        mi = mi_s[...] - md * ed * INV_SQRT_NEIGH
        ph = _silu_h(jnp.dot(mi, wh1a_ref[...], preferred_element_type=jnp.float32)
                     + jnp.dot(f, wh1b_ref[...], preferred_element_type=jnp.float32)
                     + bh1_ref[...])
        ph = _silu_h(jnp.dot(ph, wh2_ref[...], preferred_element_type=jnp.float32)
                     + bh2_ref[...])
        fo_ref[...] = jnp.dot(ph, who_ref[...], preferred_element_type=jnp.float32) \
            + bho_ref[...] + f


def _group_sum_matrix():
    g = np.zeros((D, V), dtype=np.float32)
    for v in range(V):
        g[v * C:(v + 1) * C, v] = 1.0
    return g


def _bdiag(a, b):
    za = jnp.zeros((a.shape[0], b.shape[1]), jnp.float32)
    zb = jnp.zeros((b.shape[0], a.shape[1]), jnp.float32)
    return jnp.concatenate(
        [jnp.concatenate([a, za], axis=1), jnp.concatenate([zb, b], axis=1)], axis=0)


_G_NP = _group_sum_matrix()
_GG_NP = _G_NP @ _G_NP.T


def _gaug_np():
    g = np.zeros((A, L), dtype=np.float32)
    g[0:D, 0:D] = -2.0 * _GG_NP
    g[D:2 * D, D:2 * D] = -2.0 * _GG_NP
    g[2 * D:3 * D, 0:D] = np.eye(D)
    g[2 * D:3 * D, D:2 * D] = np.eye(D)
    g[3 * D:5 * D, 0:2 * D] = np.eye(2 * D)
    g[5 * D:A, 2 * D:L] = np.eye(BR)
    return g


_GAUG_NP = _gaug_np()
_EYE_NP = np.eye(BR, dtype=np.float32)


@jax.jit
def kernel(node_vectors, node_features, We1, be1, We2, be2, Wx1, bx1, Wx2, bx2,
           Wxo, bxo, Winf, binf, Wh1, bh1, Wh2, bh2, Who, bho):
    xflat = node_vectors.reshape(N, D)
    G = jnp.asarray(_G_NP)
    GT = G.T
    GG = jnp.asarray(_GG_NP)

    eb = np.zeros((2, 2 * H), dtype=np.float32)
    eb[0, :H] = 1.0
    eb[1, H:] = 1.0
    eb = jnp.asarray(eb)
    w1v12 = (G @ We1[:V]) / C
    wxo12 = Wxo @ GT
    winfw = _bdiag(Winf, Winf) @ eb
    binfw = jnp.tile(binf, 2).reshape(1, 2) @ eb
    w1s = 0.5 * We1[V:V + F]
    w1r = 0.5 * We1[V + F:]

    full = lambda shape: pl.BlockSpec(shape, lambda i: (0, 0))
    blk = lambda shape: pl.BlockSpec(shape, lambda i: (i, 0))

    vec_out, features_out = pl.pallas_call(
        _body,
        grid=(NB,),
        in_specs=[
            full((N, F)),
            full((N2, 2 * F)),
            full((N, D)),
            full((N2, 2 * D)),
            blk((BR, D)),
            full((BR, BR)),
            full((F, H)),
            full((2 * F, 2 * H)),
            full((F, H)),
            full((1, H)),
            full((D, D)),
            full((2 * D, 2 * D)),
            full((2 * D, 2 * H)),
            full((2 * H, 2 * H)),
            full((1, 2 * H)),
            full((2 * H, 2 * H)),
            full((1, 2 * H)),
            full((2 * H, 2 * H)),
            full((1, 2 * H)),
            full((2 * H, 2 * D)),
            full((1, 2 * D)),
            full((2 * H, 2 * H)),
            full((1, 2 * H)),
            full((A, L)),
            full((H, H)),
            full((1, H)),
            full((H, 1)),
            full((1, 1)),
            full((H, H)),
            full((F, H)),
            full((1, H)),
            full((H, H)),
            full((1, H)),
            full((H, F)),
            full((1, F)),
        ],
        out_specs=(
            blk((BR, D)),
            full((N, F)),
        ),
        out_shape=(
            jax.ShapeDtypeStruct((N, D), jnp.float32),
            jax.ShapeDtypeStruct((N, F), jnp.float32),
        ),
        scratch_shapes=[
            pltpu.VMEM((N, H), jnp.float32),
            pltpu.VMEM((N, H), jnp.float32),
            pltpu.VMEM((N, D), jnp.float32),
            pltpu.VMEM((E2, A), jnp.float32),
            pltpu.VMEM((E2, 2 * H), jnp.float32),
            pltpu.VMEM((N, H), jnp.float32),
        ],
    )(node_features, node_features.reshape(N2, 2 * F), xflat,
      xflat.reshape(N2, 2 * D), xflat, jnp.asarray(_EYE_NP),
      w1s, _bdiag(w1s, w1s), w1r, 0.5 * be1.reshape(1, H), GG, _bdiag(GG, GG),
      0.5 * _bdiag(w1v12, w1v12), 0.5 * _bdiag(We2, We2),
      0.5 * jnp.tile(be2, 2).reshape(1, 2 * H),
      0.5 * _bdiag(Wx1, Wx1), 0.5 * jnp.tile(bx1, 2).reshape(1, 2 * H),
      0.5 * _bdiag(Wx2, Wx2), 0.5 * jnp.tile(bx2, 2).reshape(1, 2 * H),
      _bdiag(wxo12, wxo12), jnp.tile(bxo.reshape(1, V) @ GT, (1, 2)),
      0.5 * winfw, 0.5 * binfw, jnp.asarray(_GAUG_NP),
      0.5 * We2, 0.5 * be2.reshape(1, H), 0.5 * Winf, 0.5 * binf.reshape(1, 1),
      0.5 * Wh1[:H], 0.5 * Wh1[H:], 0.5 * bh1.reshape(1, H), 0.5 * Wh2,
      0.5 * bh2.reshape(1, H), Who, bho.reshape(1, F))

    return vec_out.reshape(N, V, C), features_out
